# baseline (speedup 1.0000x reference)
import functools

import jax
import jax.numpy as jnp
from jax import lax
from jax.experimental import pallas as pl
from jax.experimental.pallas import tpu as pltpu
from jax.experimental.pallas import tpu_sc as plsc

_CHUNK = 128
_P = 128
_R = 64


def _pack_body(in_ref, out_ref, *, D, kf):
    x = in_ref[...]
    ng = x.shape[1] // _P
    xr = x.reshape(D, ng, _P)
    bits = lax.bitcast_convert_type(
        xr.astype(jnp.bfloat16).astype(jnp.float32), jnp.uint32)
    lo = bits[:, :, 0:64]
    hi = bits[:, :, 64:128]
    y = lax.bitcast_convert_type(hi | (lo >> 16), jnp.float32)
    y = y.reshape(D, ng * 64)
    cg = kf * _P
    groups = []
    for r in range(_R):
        yb = y[:, r * cg:(r + 1) * cg]
        z = jnp.concatenate(
            [yb[:, k * _P:(k + 1) * _P] for k in range(kf)], axis=0)
        groups.append(z.T)
    out_ref[...] = jnp.concatenate(groups, axis=0)


@functools.lru_cache(maxsize=None)
def _make_pack(N, D):
    kf = _P // D
    cols = _R * kf * _P * 2
    grid = (N + cols - 1) // cols
    M = grid * _R * _P
    return pl.pallas_call(
        functools.partial(_pack_body, D=D, kf=kf),
        grid=(grid,),
        in_specs=[pl.BlockSpec((D, cols), lambda g: (0, g))],
        out_specs=pl.BlockSpec((_R * _P, _P), lambda g: (g, 0)),
        out_shape=jax.ShapeDtypeStruct((M, _P), jnp.float32),
    )


@functools.lru_cache(maxsize=None)
def _make_sc_gather(B):
    info = plsc.get_sparse_core_info()
    NC, NS, L = info.num_cores, info.num_subcores, info.num_lanes
    NW = NC * NS
    bpw = B // NW
    nch = bpw // _CHUNK
    mesh = plsc.VectorSubcoreMesh(core_axis_name="c", subcore_axis_name="s")

    @functools.partial(
        pl.kernel,
        mesh=mesh,
        out_type=[jax.ShapeDtypeStruct((B, _P), jnp.float32) for _ in range(4)],
        scratch_types=[
            pltpu.VMEM((nch, _CHUNK), jnp.int32),
            pltpu.VMEM((nch, _CHUNK), jnp.int32),
            pltpu.VMEM((nch, _CHUNK), jnp.int32),
            pltpu.VMEM((nch, _CHUNK), jnp.int32),
            pltpu.VMEM((nch, _CHUNK), jnp.int32),
            pltpu.VMEM((nch, _CHUNK), jnp.int32),
            pltpu.VMEM((_CHUNK, _P), jnp.float32),
            pltpu.VMEM((_CHUNK, _P), jnp.float32),
            pltpu.VMEM((_CHUNK, _P), jnp.float32),
            pltpu.VMEM((_CHUNK, _P), jnp.float32),
            pltpu.SemaphoreType.DMA,
        ],
    )
    def gather(user_hbm, item_hbm, ug_hbm, ig_hbm, um_hbm, im_hbm,
               out_ug, out_ig, out_um, out_im,
               uidx, iidx, ug_i, ig_i, um_i, im_i,
               ug_v, ig_v, um_v, im_v, sem):
        wid = lax.axis_index("s") * NC + lax.axis_index("c")
        pltpu.sync_copy(user_hbm.at[pl.ds(wid * nch, nch)], uidx)
        pltpu.sync_copy(item_hbm.at[pl.ds(wid * nch, nch)], iidx)
        for c in range(nch):
            for j in range(_CHUNK // L):
                sl = pl.ds(j * L, L)
                u = uidx[c, sl]
                i = iidx[c, sl]
                ju = ((u >> 7) << 6) + (u & 63)
                ji = ((i >> 7) << 6) + (i & 63)
                lo_u = ju & 127
                lo_i = ji & 127
                ug_i[c, sl] = ((ju >> 9) << 7) + lo_u
                ig_i[c, sl] = ((ji >> 9) << 7) + lo_i
                um_i[c, sl] = ((ju >> 8) << 7) + lo_u
                im_i[c, sl] = ((ji >> 8) << 7) + lo_i
        for c in range(nch):
            c1 = pltpu.async_copy(ug_hbm.at[ug_i.at[c]], ug_v, sem)
            c2 = pltpu.async_copy(ig_hbm.at[ig_i.at[c]], ig_v, sem)
            c3 = pltpu.async_copy(um_hbm.at[um_i.at[c]], um_v, sem)
            c4 = pltpu.async_copy(im_hbm.at[im_i.at[c]], im_v, sem)
            out_sl = pl.ds(wid * bpw + c * _CHUNK, _CHUNK)
            c1.wait()
            pltpu.sync_copy(ug_v, out_ug.at[out_sl])
            c2.wait()
            pltpu.sync_copy(ig_v, out_ig.at[out_sl])
            c3.wait()
            pltpu.sync_copy(um_v, out_um.at[out_sl])
            c4.wait()
            pltpu.sync_copy(im_v, out_im.at[out_sl])

    return gather


def _mlp_body(u_ref, i_ref, ug, ig, um, im, W1u, W1i, b1, W2, b2, Wog, Wom, bo, out):
    u = u_ref[...]
    i = i_ref[...]
    ju = ((u >> 7) << 6) + (u & 63)
    ji = ((i >> 7) << 6) + (i & 63)
    u4 = (ju >> 7) & 3
    i4 = (ji >> 7) & 3
    uh = (u >> 6) & 1
    ih = (i >> 6) & 1

    def sel4(v, s):
        lo = jnp.where(s == 0, v[:, 0:32], v[:, 32:64])
        hi = jnp.where(s == 2, v[:, 64:96], v[:, 96:128])
        return jnp.where(s < 2, lo, hi)

    def sel2(v, s):
        return jnp.where(s == 0, v[:, 0:64], v[:, 64:128])

    def unpack(v, h):
        bits = lax.bitcast_convert_type(v, jnp.uint32)
        lo = lax.bitcast_convert_type(bits << 16, jnp.float32)
        hi = lax.bitcast_convert_type(bits & jnp.uint32(0xFFFF0000), jnp.float32)
        return jnp.where(h == 0, lo, hi)

    gu = unpack(sel4(ug[...], u4), uh)
    gi = unpack(sel4(ig[...], i4), ih)
    mu = unpack(sel2(um[...], u4 & 1), uh)
    mi = unpack(sel2(im[...], i4 & 1), ih)

    h = jnp.dot(mu, W1u[...], preferred_element_type=jnp.float32)
    h += jnp.dot(mi, W1i[...], preferred_element_type=jnp.float32)
    h = jnp.maximum(h + b1[...], 0.0)
    m = jnp.dot(h, W2[...], preferred_element_type=jnp.float32)
    m = jnp.maximum(m + b2[...], 0.0)
    g = gu * gi
    out[...] = (jnp.dot(g, Wog[...], preferred_element_type=jnp.float32)
                + jnp.dot(m, Wom[...], preferred_element_type=jnp.float32)
                + bo[...])


@functools.lru_cache(maxsize=None)
def _make_tc_mlp(B, Dg, Dm, H1, H2, bs):
    grid = B // bs
    row = lambda g: (g, 0)
    rep = lambda g: (0, 0)
    return pl.pallas_call(
        _mlp_body,
        grid=(grid,),
        in_specs=[
            pl.BlockSpec((bs, 1), row),
            pl.BlockSpec((bs, 1), row),
            pl.BlockSpec((bs, _P), row),
            pl.BlockSpec((bs, _P), row),
            pl.BlockSpec((bs, _P), row),
            pl.BlockSpec((bs, _P), row),
            pl.BlockSpec((Dm, H1), rep),
            pl.BlockSpec((Dm, H1), rep),
            pl.BlockSpec((1, H1), rep),
            pl.BlockSpec((H1, H2), rep),
            pl.BlockSpec((1, H2), rep),
            pl.BlockSpec((Dg, 1), rep),
            pl.BlockSpec((H2, 1), rep),
            pl.BlockSpec((1, 1), rep),
        ],
        out_specs=pl.BlockSpec((bs, 1), row),
        out_shape=jax.ShapeDtypeStruct((B, 1), jnp.float32),
    )


def kernel(user, item, embed_user_gmf, embed_item_gmf, embed_user_mlp, embed_item_mlp,
           W1, b1, W2, b2, Wo, bo):
    B = user.shape[0]
    Dg = embed_user_gmf.shape[1]
    Dm = embed_user_mlp.shape[1]
    H1 = W1.shape[1]
    H2 = W2.shape[1]

    ug_p = _make_pack(embed_user_gmf.shape[0], Dg)(embed_user_gmf.T)
    ig_p = _make_pack(embed_item_gmf.shape[0], Dg)(embed_item_gmf.T)
    um_p = _make_pack(embed_user_mlp.shape[0], Dm)(embed_user_mlp.T)
    im_p = _make_pack(embed_item_mlp.shape[0], Dm)(embed_item_mlp.T)

    user_i = user.astype(jnp.int32)
    item_i = item.astype(jnp.int32)
    user2 = user_i.reshape(B // _CHUNK, _CHUNK)
    item2 = item_i.reshape(B // _CHUNK, _CHUNK)

    gather = _make_sc_gather(B)
    ug, ig, um, im = gather(user2, item2, ug_p, ig_p, um_p, im_p)

    mlp = _make_tc_mlp(B, Dg, Dm, H1, H2, bs=2048)
    logit = mlp(user_i.reshape(B, 1), item_i.reshape(B, 1), ug, ig, um, im,
                W1[:Dm], W1[Dm:], b1.reshape(1, H1),
                W2, b2.reshape(1, H2),
                Wo[:Dg], Wo[Dg:], bo.reshape(1, 1))
    return logit.reshape(B)


# speedup vs baseline: 1.1447x; 1.1447x over previous
import functools

import jax
import jax.numpy as jnp
from jax import lax
from jax.experimental import pallas as pl
from jax.experimental.pallas import tpu as pltpu
from jax.experimental.pallas import tpu_sc as plsc

_CHUNK = 128
_P = 128
_R = 128


def _pack_body(in_ref, out_ref, *, D, kf):
    x = in_ref[...]
    cg = kf * _P
    groups = []
    for r in range(_R):
        xb = x[:, r * cg:(r + 1) * cg]
        z = jnp.concatenate(
            [xb[:, k * _P:(k + 1) * _P] for k in range(kf)], axis=0)
        groups.append(z.T)
    out_ref[...] = jnp.concatenate(groups, axis=0)


@functools.lru_cache(maxsize=None)
def _make_pack(N, D):
    kf = _P // D
    cols = _R * kf * _P
    grid = (N + cols - 1) // cols
    M = grid * _R * _P
    return pl.pallas_call(
        functools.partial(_pack_body, D=D, kf=kf),
        grid=(grid,),
        in_specs=[pl.BlockSpec((D, cols), lambda g: (0, g))],
        out_specs=pl.BlockSpec((_R * _P, _P), lambda g: (g, 0)),
        out_shape=jax.ShapeDtypeStruct((M, _P), jnp.float32),
    )


@functools.lru_cache(maxsize=None)
def _make_sc_gather(B):
    info = plsc.get_sparse_core_info()
    NC, NS, L = info.num_cores, info.num_subcores, info.num_lanes
    NW = NC * NS
    bpw = B // NW
    nch = bpw // _CHUNK
    mesh = plsc.VectorSubcoreMesh(core_axis_name="c", subcore_axis_name="s")

    @functools.partial(
        pl.kernel,
        mesh=mesh,
        out_type=[jax.ShapeDtypeStruct((B, _P), jnp.float32) for _ in range(4)],
        scratch_types=[
            pltpu.VMEM((nch, _CHUNK), jnp.int32),
            pltpu.VMEM((nch, _CHUNK), jnp.int32),
            pltpu.VMEM((nch, _CHUNK), jnp.int32),
            pltpu.VMEM((nch, _CHUNK), jnp.int32),
            pltpu.VMEM((nch, _CHUNK), jnp.int32),
            pltpu.VMEM((nch, _CHUNK), jnp.int32),
            pltpu.VMEM((_CHUNK, _P), jnp.float32),
            pltpu.VMEM((_CHUNK, _P), jnp.float32),
            pltpu.VMEM((_CHUNK, _P), jnp.float32),
            pltpu.VMEM((_CHUNK, _P), jnp.float32),
            pltpu.SemaphoreType.DMA,
        ],
    )
    def gather(user_hbm, item_hbm, ug_hbm, ig_hbm, um_hbm, im_hbm,
               out_ug, out_ig, out_um, out_im,
               uidx, iidx, ug_i, ig_i, um_i, im_i,
               ug_v, ig_v, um_v, im_v, sem):
        wid = lax.axis_index("s") * NC + lax.axis_index("c")
        pltpu.sync_copy(user_hbm.at[pl.ds(wid * nch, nch)], uidx)
        pltpu.sync_copy(item_hbm.at[pl.ds(wid * nch, nch)], iidx)
        for c in range(nch):
            for j in range(_CHUNK // L):
                sl = pl.ds(j * L, L)
                u = uidx[c, sl]
                i = iidx[c, sl]
                lo_u = u & 127
                lo_i = i & 127
                ug_i[c, sl] = ((u >> 9) << 7) + lo_u
                ig_i[c, sl] = ((i >> 9) << 7) + lo_i
                um_i[c, sl] = ((u >> 8) << 7) + lo_u
                im_i[c, sl] = ((i >> 8) << 7) + lo_i
        for c in range(nch):
            c1 = pltpu.async_copy(ug_hbm.at[ug_i.at[c]], ug_v, sem)
            c2 = pltpu.async_copy(ig_hbm.at[ig_i.at[c]], ig_v, sem)
            c3 = pltpu.async_copy(um_hbm.at[um_i.at[c]], um_v, sem)
            c4 = pltpu.async_copy(im_hbm.at[im_i.at[c]], im_v, sem)
            out_sl = pl.ds(wid * bpw + c * _CHUNK, _CHUNK)
            c1.wait()
            pltpu.sync_copy(ug_v, out_ug.at[out_sl])
            c2.wait()
            pltpu.sync_copy(ig_v, out_ig.at[out_sl])
            c3.wait()
            pltpu.sync_copy(um_v, out_um.at[out_sl])
            c4.wait()
            pltpu.sync_copy(im_v, out_im.at[out_sl])

    return gather


def _mlp_body(u_ref, i_ref, ug, ig, um, im, W1u, W1i, b1, W2, b2, Wog, Wom, bo, out):
    u4 = (u_ref[...] >> 7) & 3
    i4 = (i_ref[...] >> 7) & 3
    u2 = u4 & 1
    i2 = i4 & 1

    def sel4(v, s):
        lo = jnp.where(s == 0, v[:, 0:32], v[:, 32:64])
        hi = jnp.where(s == 2, v[:, 64:96], v[:, 96:128])
        return jnp.where(s < 2, lo, hi)

    def sel2(v, s):
        return jnp.where(s == 0, v[:, 0:64], v[:, 64:128])

    gu = sel4(ug[...], u4)
    gi = sel4(ig[...], i4)
    mu = sel2(um[...], u2)
    mi = sel2(im[...], i2)

    h = jnp.dot(mu, W1u[...], preferred_element_type=jnp.float32)
    h += jnp.dot(mi, W1i[...], preferred_element_type=jnp.float32)
    h = jnp.maximum(h + b1[...], 0.0)
    m = jnp.dot(h, W2[...], preferred_element_type=jnp.float32)
    m = jnp.maximum(m + b2[...], 0.0)
    g = gu * gi
    out[...] = (jnp.dot(g, Wog[...], preferred_element_type=jnp.float32)
                + jnp.dot(m, Wom[...], preferred_element_type=jnp.float32)
                + bo[...])


@functools.lru_cache(maxsize=None)
def _make_tc_mlp(B, Dg, Dm, H1, H2, bs):
    grid = B // bs
    row = lambda g: (g, 0)
    rep = lambda g: (0, 0)
    return pl.pallas_call(
        _mlp_body,
        grid=(grid,),
        in_specs=[
            pl.BlockSpec((bs, 1), row),
            pl.BlockSpec((bs, 1), row),
            pl.BlockSpec((bs, _P), row),
            pl.BlockSpec((bs, _P), row),
            pl.BlockSpec((bs, _P), row),
            pl.BlockSpec((bs, _P), row),
            pl.BlockSpec((Dm, H1), rep),
            pl.BlockSpec((Dm, H1), rep),
            pl.BlockSpec((1, H1), rep),
            pl.BlockSpec((H1, H2), rep),
            pl.BlockSpec((1, H2), rep),
            pl.BlockSpec((Dg, 1), rep),
            pl.BlockSpec((H2, 1), rep),
            pl.BlockSpec((1, 1), rep),
        ],
        out_specs=pl.BlockSpec((bs, 1), row),
        out_shape=jax.ShapeDtypeStruct((B, 1), jnp.float32),
    )


def kernel(user, item, embed_user_gmf, embed_item_gmf, embed_user_mlp, embed_item_mlp,
           W1, b1, W2, b2, Wo, bo):
    B = user.shape[0]
    Dg = embed_user_gmf.shape[1]
    Dm = embed_user_mlp.shape[1]
    H1 = W1.shape[1]
    H2 = W2.shape[1]

    ug_p = _make_pack(embed_user_gmf.shape[0], Dg)(embed_user_gmf.T)
    ig_p = _make_pack(embed_item_gmf.shape[0], Dg)(embed_item_gmf.T)
    um_p = _make_pack(embed_user_mlp.shape[0], Dm)(embed_user_mlp.T)
    im_p = _make_pack(embed_item_mlp.shape[0], Dm)(embed_item_mlp.T)

    user_i = user.astype(jnp.int32)
    item_i = item.astype(jnp.int32)
    user2 = user_i.reshape(B // _CHUNK, _CHUNK)
    item2 = item_i.reshape(B // _CHUNK, _CHUNK)

    gather = _make_sc_gather(B)
    ug, ig, um, im = gather(user2, item2, ug_p, ig_p, um_p, im_p)

    mlp = _make_tc_mlp(B, Dg, Dm, H1, H2, bs=2048)
    logit = mlp(user_i.reshape(B, 1), item_i.reshape(B, 1), ug, ig, um, im,
                W1[:Dm], W1[Dm:], b1.reshape(1, H1),
                W2, b2.reshape(1, H2),
                Wo[:Dg], Wo[Dg:], bo.reshape(1, 1))
    return logit.reshape(B)
